# baseline (speedup 1.0000x reference)
import functools

import jax
import jax.numpy as jnp
from jax import lax
from jax.experimental import pallas as pl
from jax.experimental.pallas import tpu as pltpu
from jax.experimental.pallas import tpu_sc as plsc

N = 5000
NPAD = 5120
D = 128
NE = 160000
NEPAD = 163840
NC = 2
NS = 16
NW = NC * NS
CH = 64
NCH = NEPAD // (NW * CH)
RT = NPAD // NS
BLK = 128
GRID = NPAD // BLK

_f32 = jnp.float32


@functools.lru_cache(maxsize=None)
def _sc_mesh():
    return plsc.VectorSubcoreMesh(core_axis_name="c", subcore_axis_name="s")


def _hist_body(su_h, du_h, si_h, di_h, ones_h, zeros_h,
               o_su, o_du, o_si, o_di,
               su_v, du_v, si_v, di_v, ones_v,
               sem_0, sem_1,
               h_s):
    cid = lax.axis_index("c")
    sid = lax.axis_index("s")
    wid = cid * NS + sid
    pltpu.sync_copy(su_h.at[wid], su_v)
    pltpu.sync_copy(du_h.at[wid], du_v)
    pltpu.sync_copy(si_h.at[wid], si_v)
    pltpu.sync_copy(di_h.at[wid], di_v)
    pltpu.sync_copy(ones_h, ones_v)
    sems = (sem_0, sem_1)
    for idx_v, out in ((su_v, o_su), (du_v, o_du), (si_v, o_si), (di_v, o_di)):
        pltpu.sync_copy(zeros_h, h_s.at[pl.ds(sid * RT, RT)])
        plsc.subcore_barrier()

        @pl.loop(0, NCH // 2)
        def _outer(t, idx_v=idx_v):
            for p in range(2):
                j = t * 2 + p

                @pl.when(j >= 2)
                def _drain(p=p, j=j, idx_v=idx_v):
                    pltpu.make_async_copy(
                        ones_v, h_s.at[idx_v.at[j - 2]], sems[p]).wait()

                pltpu.async_copy(ones_v, h_s.at[idx_v.at[j]], sems[p],
                                 add=True)

        for p in range(2):
            pltpu.make_async_copy(
                ones_v, h_s.at[idx_v.at[NCH - 2 + p]], sems[p]).wait()
        plsc.subcore_barrier()
        base = cid * NPAD + sid * RT
        pltpu.sync_copy(h_s.at[pl.ds(sid * RT, RT)], out.at[pl.ds(base, RT)])
        plsc.subcore_barrier()


@functools.lru_cache(maxsize=None)
def _hist_kernel():
    return pl.kernel(
        _hist_body,
        out_type=[jax.ShapeDtypeStruct((NC * NPAD, D), _f32)] * 4,
        mesh=_sc_mesh(),
        scratch_types=(
            [pltpu.VMEM((NCH, CH), jnp.int32)] * 4
            + [pltpu.VMEM((CH, D), _f32)]
            + [pltpu.SemaphoreType.DMA] * 2
            + [pltpu.VMEM_SHARED((NPAD, D), _f32)]
        ),
    )


def _hist_call(*args):
    return _hist_kernel()(*args)


def _scatter_body(xnu_h, xni_h, su_h, du_h, si_h, di_h, zeros_h,
                  o_aggi, o_aggu,
                  su_v, du_v, si_v, di_v,
                  buf_0, buf_1, buf_2, buf_3, buf_4,
                  sg_0, sg_1, sg_2, sg_3, sg_4,
                  sa_0, sa_1, sa_2, sa_3, sa_4,
                  agg_s):
    cid = lax.axis_index("c")
    sid = lax.axis_index("s")
    wid = cid * NS + sid
    pltpu.sync_copy(su_h.at[wid], su_v)
    pltpu.sync_copy(du_h.at[wid], du_v)
    pltpu.sync_copy(si_h.at[wid], si_v)
    pltpu.sync_copy(di_h.at[wid], di_v)
    base = cid * NPAD + sid * RT
    bufs = (buf_0, buf_1, buf_2, buf_3, buf_4)
    sg = (sg_0, sg_1, sg_2, sg_3, sg_4)
    sa = (sa_0, sa_1, sa_2, sa_3, sa_4)
    for src_h, sidx_v, didx_v, out in (
        (xnu_h, su_v, du_v, o_aggi),
        (xni_h, si_v, di_v, o_aggu),
    ):
        pltpu.sync_copy(zeros_h, agg_s.at[pl.ds(sid * RT, RT)])
        plsc.subcore_barrier()
        for p0 in range(2):
            pltpu.async_copy(src_h.at[sidx_v.at[p0]], bufs[p0], sg[p0])

        @pl.loop(0, NCH // 5)
        def _outer(t, src_h=src_h, sidx_v=sidx_v, didx_v=didx_v):
            for p in range(5):
                j = t * 5 + p
                q = (p + 2) % 5

                @pl.when(j >= 3)
                def _drain_add(j=j, q=q, didx_v=didx_v):
                    pltpu.make_async_copy(
                        bufs[q], agg_s.at[didx_v.at[j - 3]], sa[q]).wait()

                @pl.when(j + 2 < NCH)
                def _gather(j=j, q=q, src_h=src_h, sidx_v=sidx_v):
                    pltpu.async_copy(src_h.at[sidx_v.at[j + 2]], bufs[q], sg[q])

                pltpu.make_async_copy(
                    src_h.at[sidx_v.at[j]], bufs[p], sg[p]).wait()
                pltpu.async_copy(bufs[p], agg_s.at[didx_v.at[j]], sa[p],
                                 add=True)

        for pe in range(NCH - 3, NCH):
            pltpu.make_async_copy(
                bufs[pe % 5], agg_s.at[didx_v.at[pe]], sa[pe % 5]).wait()
        plsc.subcore_barrier()
        pltpu.sync_copy(agg_s.at[pl.ds(sid * RT, RT)], out.at[pl.ds(base, RT)])
        plsc.subcore_barrier()


@functools.lru_cache(maxsize=None)
def _scatter_kernel():
    return pl.kernel(
        _scatter_body,
        out_type=[jax.ShapeDtypeStruct((NC * NPAD, D), _f32)] * 2,
        mesh=_sc_mesh(),
        scratch_types=(
            [pltpu.VMEM((NCH, CH), jnp.int32)] * 4
            + [pltpu.VMEM((CH, D), _f32)] * 5
            + [pltpu.SemaphoreType.DMA] * 10
            + [pltpu.VMEM_SHARED((NPAD, D), _f32)]
        ),
    )


def _scatter_call(*args):
    return _scatter_kernel()(*args)


def _scatter_emu(xnu, xni, su, du, si, di, zeros_d):
    def one(x, s, d):
        parts = []
        for c in range(2):
            ss = s.reshape(NW, -1)[c * NS:(c + 1) * NS].reshape(-1)
            dd = d.reshape(NW, -1)[c * NS:(c + 1) * NS].reshape(-1)
            parts.append(jnp.zeros((NPAD, D), _f32).at[dd].add(x[ss]))
        return jnp.concatenate(parts, 0)
    return one(xnu, su, du), one(xni, si, di)


def _hist_emu(su, du, si, di, ones_h, zeros_h):
    outs = []
    for idx in (su, du, si, di):
        flat = idx.reshape(NW, -1)
        res = []
        for c in range(2):
            part = flat[c * NS:(c + 1) * NS].reshape(-1)
            h = jnp.zeros((NPAD,), _f32).at[part].add(1.0)
            res.append(jnp.broadcast_to(h[:, None], (NPAD, 16)))
        outs.append(jnp.concatenate(res, 0))
    return outs


def _ln(x, g, b):
    mu = jnp.mean(x, axis=-1, keepdims=True)
    xc = x - mu
    var = jnp.mean(xc * xc, axis=-1, keepdims=True)
    return xc * lax.rsqrt(var + 1e-5) * g + b


def _prep_body(xu, xi, hsu0, hsu1, hdu0, hdu1, hsi0, hsi1, hdi0, hdi1,
               xnu, xni, dsu, ddu, dsi, ddi):
    def dn(h0, h1):
        return lax.rsqrt(jnp.maximum(h0[:, 0:1] + h1[:, 0:1], 1.0))

    a = dn(hsu0[...], hsu1[...])
    c = dn(hsi0[...], hsi1[...])
    dsu[...] = a
    ddu[...] = dn(hdu0[...], hdu1[...])
    dsi[...] = c
    ddi[...] = dn(hdi0[...], hdi1[...])
    xnu[...] = xu[...] * a
    xni[...] = xi[...] * c


def _tc_prep(xu, xi, hsu, hdu, hsi, hdi):
    row = pl.BlockSpec((BLK, D), lambda i: (i, 0))
    hist = pl.BlockSpec((BLK, D), lambda i: (i, 0))
    col = pl.BlockSpec((BLK, 1), lambda i: (i, 0))
    hists = []
    for h in (hsu, hdu, hsi, hdi):
        hists += [h[:NPAD], h[NPAD:]]
    return pl.pallas_call(
        _prep_body,
        grid=(GRID,),
        in_specs=[row, row] + [hist] * 8,
        out_specs=[row, row, col, col, col, col],
        out_shape=[jax.ShapeDtypeStruct((NPAD, D), _f32)] * 2
        + [jax.ShapeDtypeStruct((NPAD, 1), _f32)] * 4,
    )(xu, xi, *hists)


def _layer_body(act, scale_out,
                ai0, ai1, au0, au1, wui, wiu, bui, biu,
                gi, bei, gu, beu, ddu, ddi, dsu, dsi,
                out_i, out_u):
    hi = jnp.dot(ai0[...] + ai1[...], wui[...],
                 preferred_element_type=_f32,
                 precision=lax.Precision.HIGHEST) * ddu[...] + bui[...]
    hu = jnp.dot(au0[...] + au1[...], wiu[...],
                 preferred_element_type=_f32,
                 precision=lax.Precision.HIGHEST) * ddi[...] + biu[...]
    if act:
        hi = jnp.maximum(hi, 0.0)
        hu = jnp.maximum(hu, 0.0)
    hi = _ln(hi, gi[...], bei[...])
    hu = _ln(hu, gu[...], beu[...])
    if scale_out:
        hi = hi * dsi[...]
        hu = hu * dsu[...]
    out_i[...] = hi
    out_u[...] = hu


def _tc_layer(act, scale_out, aggi, aggu, wui, wiu, bui, biu,
              gi, bei, gu, beu, ddu, ddi, dsu, dsi):
    row = pl.BlockSpec((BLK, D), lambda i: (i, 0))
    full = pl.BlockSpec((D, D), lambda i: (0, 0))
    vec = pl.BlockSpec((1, D), lambda i: (0, 0))
    col = pl.BlockSpec((BLK, 1), lambda i: (i, 0))
    return pl.pallas_call(
        functools.partial(_layer_body, act, scale_out),
        grid=(GRID,),
        in_specs=[row] * 4 + [full] * 2 + [vec] * 6 + [col] * 4,
        out_specs=[row, row],
        out_shape=[jax.ShapeDtypeStruct((NPAD, D), _f32)] * 2,
    )(aggi[:NPAD], aggi[NPAD:], aggu[:NPAD], aggu[NPAD:],
      wui, wiu, bui, biu, gi, bei, gu, beu, ddu, ddi, dsu, dsi)


def _prep_idx(e):
    pad = N + (jnp.arange(NEPAD - NE, dtype=jnp.int32) % (NPAD - N))
    out = []
    for r in range(2):
        v = jnp.concatenate([e[r].astype(jnp.int32), pad])
        out.append(v.reshape(NW, NCH, CH))
    return out


def kernel(x_user, x_item, eidx_ui, eidx_iu,
           W0_ui, b0_ui, W0_iu, b0_iu, g0_u, be0_u, g0_i, be0_i,
           W1_ui, b1_ui, W1_iu, b1_iu, g1_u, be1_u, g1_i, be1_i):
    su, du = _prep_idx(eidx_ui)
    si, di = _prep_idx(eidx_iu)
    zpad = jnp.zeros((NPAD - N, D), _f32)
    xu = jnp.concatenate([x_user, zpad])
    xi = jnp.concatenate([x_item, zpad])
    ones_h = jnp.ones((CH, D), _f32)
    zeros_d = jnp.zeros((RT, D), _f32)

    h_su, h_du, h_si, h_di = _hist_call(su, du, si, di, ones_h, zeros_d)
    xn_u, xn_i, dsu, ddu, dsi, ddi = _tc_prep(xu, xi, h_su, h_du, h_si, h_di)

    r2 = lambda v: v.reshape(1, -1)
    aggi, aggu = _scatter_call(xn_u, xn_i, su, du, si, di, zeros_d)
    xn1_i, xn1_u = _tc_layer(
        True, True, aggi, aggu, W0_ui, W0_iu, r2(b0_ui), r2(b0_iu),
        r2(g0_i), r2(be0_i), r2(g0_u), r2(be0_u), ddu, ddi, dsu, dsi)

    aggi2, aggu2 = _scatter_call(xn1_u, xn1_i, su, du, si, di, zeros_d)
    hi2, hu2 = _tc_layer(
        False, False, aggi2, aggu2, W1_ui, W1_iu, r2(b1_ui), r2(b1_iu),
        r2(g1_i), r2(be1_i), r2(g1_u), r2(be1_u), ddu, ddi, dsu, dsi)

    return hu2[:N], hi2[:N]


# speedup vs baseline: 1.0319x; 1.0319x over previous
import functools

import jax
import jax.numpy as jnp
from jax import lax
from jax.experimental import pallas as pl
from jax.experimental.pallas import tpu as pltpu
from jax.experimental.pallas import tpu_sc as plsc

N = 5000
NPAD = 5120
D = 128
NE = 160000
NEPAD = 163840
NC = 2
NS = 16
NW = NC * NS
CH = 128
NCH = NEPAD // (NW * CH)
RT = NPAD // NS
BLK = 128
GRID = NPAD // BLK

_f32 = jnp.float32


@functools.lru_cache(maxsize=None)
def _sc_mesh():
    return plsc.VectorSubcoreMesh(core_axis_name="c", subcore_axis_name="s")


def _hist_body(su_h, du_h, si_h, di_h, ones_h, zeros_h,
               o_su, o_du, o_si, o_di,
               su_v, du_v, si_v, di_v, ones_v,
               sem_0, sem_1,
               h_s):
    cid = lax.axis_index("c")
    sid = lax.axis_index("s")
    wid = cid * NS + sid
    pltpu.sync_copy(su_h.at[wid], su_v)
    pltpu.sync_copy(du_h.at[wid], du_v)
    pltpu.sync_copy(si_h.at[wid], si_v)
    pltpu.sync_copy(di_h.at[wid], di_v)
    pltpu.sync_copy(ones_h, ones_v)
    sems = (sem_0, sem_1)
    for idx_v, out in ((su_v, o_su), (du_v, o_du), (si_v, o_si), (di_v, o_di)):
        pltpu.sync_copy(zeros_h, h_s.at[pl.ds(sid * RT, RT)])
        plsc.subcore_barrier()

        @pl.loop(0, NCH // 2)
        def _outer(t, idx_v=idx_v):
            for p in range(2):
                j = t * 2 + p

                @pl.when(j >= 2)
                def _drain(p=p, j=j, idx_v=idx_v):
                    pltpu.make_async_copy(
                        ones_v, h_s.at[idx_v.at[j - 2]], sems[p]).wait()

                pltpu.async_copy(ones_v, h_s.at[idx_v.at[j]], sems[p],
                                 add=True)

        for p in range(2):
            pltpu.make_async_copy(
                ones_v, h_s.at[idx_v.at[NCH - 2 + p]], sems[p]).wait()
        plsc.subcore_barrier()
        base = cid * NPAD + sid * RT
        pltpu.sync_copy(h_s.at[pl.ds(sid * RT, RT)], out.at[pl.ds(base, RT)])
        plsc.subcore_barrier()


@functools.lru_cache(maxsize=None)
def _hist_kernel():
    return pl.kernel(
        _hist_body,
        out_type=[jax.ShapeDtypeStruct((NC * NPAD, D), _f32)] * 4,
        mesh=_sc_mesh(),
        scratch_types=(
            [pltpu.VMEM((NCH, CH), jnp.int32)] * 4
            + [pltpu.VMEM((CH, D), _f32)]
            + [pltpu.SemaphoreType.DMA] * 2
            + [pltpu.VMEM_SHARED((NPAD, D), _f32)]
        ),
    )


def _hist_call(*args):
    return _hist_kernel()(*args)


def _scatter_body(xnu_h, xni_h, su_h, du_h, si_h, di_h, zeros_h,
                  o_aggi, o_aggu,
                  su_v, du_v, si_v, di_v,
                  buf_0, buf_1, buf_2, buf_3,
                  sg_0, sg_1, sg_2, sg_3,
                  sa_0, sa_1, sa_2, sa_3,
                  agg_s):
    cid = lax.axis_index("c")
    sid = lax.axis_index("s")
    wid = cid * NS + sid
    pltpu.sync_copy(su_h.at[wid], su_v)
    pltpu.sync_copy(du_h.at[wid], du_v)
    pltpu.sync_copy(si_h.at[wid], si_v)
    pltpu.sync_copy(di_h.at[wid], di_v)
    base = cid * NPAD + sid * RT
    bufs = (buf_0, buf_1, buf_2, buf_3)
    sg = (sg_0, sg_1, sg_2, sg_3)
    sa = (sa_0, sa_1, sa_2, sa_3)
    for src_h, sidx_v, didx_v, out in (
        (xnu_h, su_v, du_v, o_aggi),
        (xni_h, si_v, di_v, o_aggu),
    ):
        pltpu.sync_copy(zeros_h, agg_s.at[pl.ds(sid * RT, RT)])
        plsc.subcore_barrier()
        for p0 in range(2):
            pltpu.async_copy(src_h.at[sidx_v.at[p0]], bufs[p0], sg[p0])

        @pl.loop(0, NCH // 4)
        def _outer(t, src_h=src_h, sidx_v=sidx_v, didx_v=didx_v):
            for p in range(4):
                j = t * 4 + p
                q = (p + 2) % 4

                @pl.when(j >= 2)
                def _drain_add(j=j, q=q, didx_v=didx_v):
                    pltpu.make_async_copy(
                        bufs[q], agg_s.at[didx_v.at[j - 2]], sa[q]).wait()

                @pl.when(j + 2 < NCH)
                def _gather(j=j, q=q, src_h=src_h, sidx_v=sidx_v):
                    pltpu.async_copy(src_h.at[sidx_v.at[j + 2]], bufs[q], sg[q])

                pltpu.make_async_copy(
                    src_h.at[sidx_v.at[j]], bufs[p], sg[p]).wait()
                pltpu.async_copy(bufs[p], agg_s.at[didx_v.at[j]], sa[p],
                                 add=True)

        for pe in range(NCH - 2, NCH):
            pltpu.make_async_copy(
                bufs[pe % 4], agg_s.at[didx_v.at[pe]], sa[pe % 4]).wait()
        plsc.subcore_barrier()
        pltpu.sync_copy(agg_s.at[pl.ds(sid * RT, RT)], out.at[pl.ds(base, RT)])
        plsc.subcore_barrier()


@functools.lru_cache(maxsize=None)
def _scatter_kernel():
    return pl.kernel(
        _scatter_body,
        out_type=[jax.ShapeDtypeStruct((NC * NPAD, D), _f32)] * 2,
        mesh=_sc_mesh(),
        scratch_types=(
            [pltpu.VMEM((NCH, CH), jnp.int32)] * 4
            + [pltpu.VMEM((CH, D), _f32)] * 4
            + [pltpu.SemaphoreType.DMA] * 8
            + [pltpu.VMEM_SHARED((NPAD, D), _f32)]
        ),
    )


def _scatter_call(*args):
    return _scatter_kernel()(*args)


def _scatter_emu(xnu, xni, su, du, si, di, zeros_d):
    def one(x, s, d):
        parts = []
        for c in range(2):
            ss = s.reshape(NW, -1)[c * NS:(c + 1) * NS].reshape(-1)
            dd = d.reshape(NW, -1)[c * NS:(c + 1) * NS].reshape(-1)
            parts.append(jnp.zeros((NPAD, D), _f32).at[dd].add(x[ss]))
        return jnp.concatenate(parts, 0)
    return one(xnu, su, du), one(xni, si, di)


def _hist_emu(su, du, si, di, ones_h, zeros_h):
    outs = []
    for idx in (su, du, si, di):
        flat = idx.reshape(NW, -1)
        res = []
        for c in range(2):
            part = flat[c * NS:(c + 1) * NS].reshape(-1)
            h = jnp.zeros((NPAD,), _f32).at[part].add(1.0)
            res.append(jnp.broadcast_to(h[:, None], (NPAD, 16)))
        outs.append(jnp.concatenate(res, 0))
    return outs


def _ln(x, g, b):
    mu = jnp.mean(x, axis=-1, keepdims=True)
    xc = x - mu
    var = jnp.mean(xc * xc, axis=-1, keepdims=True)
    return xc * lax.rsqrt(var + 1e-5) * g + b


def _prep_body(xu, xi, hsu0, hsu1, hdu0, hdu1, hsi0, hsi1, hdi0, hdi1,
               xnu, xni, dsu, ddu, dsi, ddi):
    def dn(h0, h1):
        return lax.rsqrt(jnp.maximum(h0[:, 0:1] + h1[:, 0:1], 1.0))

    a = dn(hsu0[...], hsu1[...])
    c = dn(hsi0[...], hsi1[...])
    dsu[...] = a
    ddu[...] = dn(hdu0[...], hdu1[...])
    dsi[...] = c
    ddi[...] = dn(hdi0[...], hdi1[...])
    xnu[...] = xu[...] * a
    xni[...] = xi[...] * c


def _tc_prep(xu, xi, hsu, hdu, hsi, hdi):
    row = pl.BlockSpec((BLK, D), lambda i: (i, 0))
    hist = pl.BlockSpec((BLK, D), lambda i: (i, 0))
    col = pl.BlockSpec((BLK, 1), lambda i: (i, 0))
    hists = []
    for h in (hsu, hdu, hsi, hdi):
        hists += [h[:NPAD], h[NPAD:]]
    return pl.pallas_call(
        _prep_body,
        grid=(GRID,),
        in_specs=[row, row] + [hist] * 8,
        out_specs=[row, row, col, col, col, col],
        out_shape=[jax.ShapeDtypeStruct((NPAD, D), _f32)] * 2
        + [jax.ShapeDtypeStruct((NPAD, 1), _f32)] * 4,
    )(xu, xi, *hists)


def _mm_body(xu, xi, wui, wiu, yu, yi):
    yu[...] = jnp.dot(xu[...], wui[...], preferred_element_type=_f32,
                      precision=lax.Precision.HIGHEST)
    yi[...] = jnp.dot(xi[...], wiu[...], preferred_element_type=_f32,
                      precision=lax.Precision.HIGHEST)


def _tc_mm(xu, xi, wui, wiu):
    row = pl.BlockSpec((BLK, D), lambda i: (i, 0))
    full = pl.BlockSpec((D, D), lambda i: (0, 0))
    return pl.pallas_call(
        _mm_body,
        grid=(GRID,),
        in_specs=[row, row, full, full],
        out_specs=[row, row],
        out_shape=[jax.ShapeDtypeStruct((NPAD, D), _f32)] * 2,
    )(xu, xi, wui, wiu)


def _mid_body(ai0, ai1, au0, au1, w1ui, w1iu, bui, biu,
              gi, bei, gu, beu, ddu, ddi, dsu, dsi,
              out_u, out_i):
    hi = jnp.maximum((ai0[...] + ai1[...]) * ddu[...] + bui[...], 0.0)
    hi = _ln(hi, gi[...], bei[...])
    hu = jnp.maximum((au0[...] + au1[...]) * ddi[...] + biu[...], 0.0)
    hu = _ln(hu, gu[...], beu[...])
    out_u[...] = jnp.dot(hu * dsu[...], w1ui[...],
                         preferred_element_type=_f32,
                         precision=lax.Precision.HIGHEST)
    out_i[...] = jnp.dot(hi * dsi[...], w1iu[...],
                         preferred_element_type=_f32,
                         precision=lax.Precision.HIGHEST)


def _tc_mid(aggi, aggu, w1ui, w1iu, bui, biu,
            gi, bei, gu, beu, ddu, ddi, dsu, dsi):
    row = pl.BlockSpec((BLK, D), lambda i: (i, 0))
    full = pl.BlockSpec((D, D), lambda i: (0, 0))
    vec = pl.BlockSpec((1, D), lambda i: (0, 0))
    col = pl.BlockSpec((BLK, 1), lambda i: (i, 0))
    return pl.pallas_call(
        _mid_body,
        grid=(GRID,),
        in_specs=[row] * 4 + [full] * 2 + [vec] * 6 + [col] * 4,
        out_specs=[row, row],
        out_shape=[jax.ShapeDtypeStruct((NPAD, D), _f32)] * 2,
    )(aggi[:NPAD], aggi[NPAD:], aggu[:NPAD], aggu[NPAD:],
      w1ui, w1iu, bui, biu, gi, bei, gu, beu, ddu, ddi, dsu, dsi)


def _final_body(ai0, ai1, au0, au1, bui, biu, gi, bei, gu, beu, ddu, ddi,
                out_i, out_u):
    hi = (ai0[...] + ai1[...]) * ddu[...] + bui[...]
    out_i[...] = _ln(hi, gi[...], bei[...])
    hu = (au0[...] + au1[...]) * ddi[...] + biu[...]
    out_u[...] = _ln(hu, gu[...], beu[...])


def _tc_final(aggi, aggu, bui, biu, gi, bei, gu, beu, ddu, ddi):
    row = pl.BlockSpec((BLK, D), lambda i: (i, 0))
    vec = pl.BlockSpec((1, D), lambda i: (0, 0))
    col = pl.BlockSpec((BLK, 1), lambda i: (i, 0))
    return pl.pallas_call(
        _final_body,
        grid=(GRID,),
        in_specs=[row] * 4 + [vec] * 6 + [col] * 2,
        out_specs=[row, row],
        out_shape=[jax.ShapeDtypeStruct((NPAD, D), _f32)] * 2,
    )(aggi[:NPAD], aggi[NPAD:], aggu[:NPAD], aggu[NPAD:],
      bui, biu, gi, bei, gu, beu, ddu, ddi)


def _prep_idx(e):
    pad = N + (jnp.arange(NEPAD - NE, dtype=jnp.int32) % (NPAD - N))
    out = []
    for r in range(2):
        v = jnp.concatenate([e[r].astype(jnp.int32), pad])
        out.append(v.reshape(NW, NCH, CH))
    return out


def kernel(x_user, x_item, eidx_ui, eidx_iu,
           W0_ui, b0_ui, W0_iu, b0_iu, g0_u, be0_u, g0_i, be0_i,
           W1_ui, b1_ui, W1_iu, b1_iu, g1_u, be1_u, g1_i, be1_i):
    su, du = _prep_idx(eidx_ui)
    si, di = _prep_idx(eidx_iu)
    zpad = jnp.zeros((NPAD - N, D), _f32)
    xu = jnp.concatenate([x_user, zpad])
    xi = jnp.concatenate([x_item, zpad])
    ones_h = jnp.ones((CH, D), _f32)
    zeros_d = jnp.zeros((RT, D), _f32)

    h_su, h_du, h_si, h_di = _hist_call(su, du, si, di, ones_h, zeros_d)
    y_u, y_i = _tc_mm(xu, xi, W0_ui, W0_iu)
    xn_u, xn_i, dsu, ddu, dsi, ddi = _tc_prep(y_u, y_i, h_su, h_du, h_si, h_di)

    r2 = lambda v: v.reshape(1, -1)
    aggi, aggu = _scatter_call(xn_u, xn_i, su, du, si, di, zeros_d)
    xn1_u, xn1_i = _tc_mid(
        aggi, aggu, W1_ui, W1_iu, r2(b0_ui), r2(b0_iu),
        r2(g0_i), r2(be0_i), r2(g0_u), r2(be0_u), ddu, ddi, dsu, dsi)

    aggi2, aggu2 = _scatter_call(xn1_u, xn1_i, su, du, si, di, zeros_d)
    hi2, hu2 = _tc_final(
        aggi2, aggu2, r2(b1_ui), r2(b1_iu),
        r2(g1_i), r2(be1_i), r2(g1_u), r2(be1_u), ddu, ddi)

    return hu2[:N], hi2[:N]
